# Initial kernel scaffold; baseline (speedup 1.0000x reference)
import jax
import jax.numpy as jnp
from jax.experimental import pallas as pl


def kernel(x, edge_attr, edge_index, Wn0, bn0, We0, be0, Wm1, bm1, Wm2, bm2, Wu, bu, Wt1, bt1, Wt2, bt2):
    raise NotImplementedError("write your pallas kernel here")


# speedup vs baseline: 3.1676x; 3.1676x over previous
import functools

import jax
import jax.numpy as jnp
from jax import lax
from jax.experimental import pallas as pl
from jax.experimental.pallas import tpu as pltpu
from jax.experimental.pallas import tpu_sc as plsc

N = 10000
E = 320000
D = 128
DE_IN = 16
DE = 64
H = 128

NC = 2
NS = 16
NW = NC * NS
EPW = E // NW
CH = 80
CPG = 5
GR = CH * CPG
NCH = EPW // CH
NG = NCH // CPG
NPAD = 10240
HR = NPAD // 2
HRP = HR + NS
RPH = HR // NS
EPT = E // NS
DEP = 128
CHQ = 80
NCHQ = EPT // CHQ
DRN = RPH // CHQ

BN = 2000
BE = 4000


@functools.cache
def _sc_mesh():
    return plsc.VectorSubcoreMesh(core_axis_name="c", subcore_axis_name="s",
                                  num_cores=NC, num_subcores=NS)


@functools.cache
def _sc_gather_kernel():
    return pl.kernel(
        _sc_gather_body,
        mesh=_sc_mesh(),
        out_type=[jax.ShapeDtypeStruct((E, H), jnp.float32),
                  jax.ShapeDtypeStruct((E, H), jnp.float32)],
        scratch_types=[[pltpu.VMEM((CH,), jnp.int32) for _ in range(CPG)],
                       pltpu.VMEM((GR, H), jnp.float32),
                       pltpu.SemaphoreType.DMA,
                       pltpu.SemaphoreType.DMA],
    )


def _sc_gather_body(ps_hbm, pd_hbm, src_hbm, dst_hbm, gs_hbm, gd_hbm,
                    idxc, buf_v, isem, sem):
    wid = lax.axis_index("s") * NC + lax.axis_index("c")
    base = wid * EPW

    def run(idx_hbm, table, out):
        def grp(g, carry):
            ics = [pltpu.async_copy(idx_hbm.at[wid, g * CPG + k], idxc[k], isem)
                   for k in range(CPG)]
            for cp in ics:
                cp.wait()
            cps = [pltpu.async_copy(table.at[idxc[k]],
                                    buf_v.at[pl.ds(k * CH, CH)], sem)
                   for k in range(CPG)]
            for cp in cps:
                cp.wait()
            pltpu.sync_copy(buf_v, out.at[pl.ds(base + g * GR, GR)])
            return carry
        lax.fori_loop(0, NG, grp, 0)

    run(src_hbm, ps_hbm, gs_hbm)
    run(dst_hbm, pd_hbm, gd_hbm)


@functools.cache
def _sc_scatter_kernel():
    return pl.kernel(
        _sc_scatter_body,
        mesh=_sc_mesh(),
        out_type=jax.ShapeDtypeStruct((NPAD, DEP), jnp.float32),
        scratch_types=[pltpu.VMEM((CHQ,), jnp.int32),
                       pltpu.VMEM((CHQ,), jnp.int32),
                       pltpu.VMEM((CHQ, DEP), jnp.float32),
                       pltpu.VMEM((CHQ, DEP), jnp.float32),
                       pltpu.VMEM_SHARED((HRP, DEP), jnp.float32),
                       pltpu.SemaphoreType.DMA,
                       pltpu.SemaphoreType.DMA],
    )


def _sc_scatter_body(efp_hbm, idx_hbm, zeros_hbm, agg_hbm, il0, il1,
                     buf0, buf1, acc_sh, isem, sem):
    cid = lax.axis_index("c")
    sid = lax.axis_index("s")
    base = sid * EPT
    pltpu.sync_copy(zeros_hbm, buf0)
    for j in range(DRN):
        pltpu.sync_copy(buf0, acc_sh.at[pl.ds(sid * RPH + j * CHQ, CHQ)])
    plsc.subcore_barrier()

    bufs = (buf0, buf1)
    ils = (il0, il1)

    def load(g, b):
        pltpu.async_copy(idx_hbm.at[cid, sid, g], ils[b], isem)
        pltpu.async_copy(efp_hbm.at[pl.ds(base + g * CHQ, CHQ)], bufs[b], sem)

    def wait_load(g, b):
        pltpu.make_async_copy(idx_hbm.at[cid, sid, g], ils[b], isem).wait()
        pltpu.make_async_copy(efp_hbm.at[pl.ds(base + g * CHQ, CHQ)],
                              bufs[b], sem).wait()

    load(0, 0)

    def pair(i, carry):
        g0 = 2 * i
        wait_load(g0, 0)
        load(g0 + 1, 1)
        pltpu.sync_copy(buf0, acc_sh.at[il0], add=True)
        wait_load(g0 + 1, 1)
        @pl.when(g0 + 2 < NCHQ)
        def _():
            load(g0 + 2, 0)
        pltpu.sync_copy(buf1, acc_sh.at[il1], add=True)
        return carry
    lax.fori_loop(0, NCHQ // 2, pair, 0)

    plsc.subcore_barrier()
    for j in range(DRN):
        pltpu.sync_copy(acc_sh.at[pl.ds(sid * RPH + j * CHQ, CHQ)], buf0)
        pltpu.sync_copy(buf0, agg_hbm.at[pl.ds(cid * HR + sid * RPH + j * CHQ,
                                               CHQ)])


def _relu(v):
    return jnp.maximum(v, 0.0)


_DPR = E // 128
_DBR = _DPR


def _dst_prep_body(d_ref, o_ref):
    c = pl.program_id(0)
    i = pl.program_id(1)
    v = d_ref[...] - c * HR
    ok = (v >= 0) & (v < HR)
    rows = lax.broadcasted_iota(jnp.int32, (_DBR, 128), 0)
    cols = lax.broadcasted_iota(jnp.int32, (_DBR, 128), 1)
    e = (i * _DBR + rows) * 128 + cols
    trash = HR + e // EPT
    o_ref[...] = jnp.where(ok, v, trash)[None]


def _dst_prep(dst2d):
    return pl.pallas_call(
        _dst_prep_body,
        grid=(NC, _DPR // _DBR),
        in_specs=[pl.BlockSpec((_DBR, 128), lambda c, i: (i, 0))],
        out_specs=pl.BlockSpec((1, _DBR, 128), lambda c, i: (c, i, 0)),
        out_shape=jax.ShapeDtypeStruct((NC, _DPR, 128), jnp.int32),
    )(dst2d)


def _dot(a, b):
    return lax.dot_general(a, b, (((1,), (0,)), ((), ())),
                           preferred_element_type=jnp.float32)


def _nodes0_body(x_ref, wn, bn, ws, wd, nf_ref, ps_ref, pd_ref):
    nf = _relu(_dot(x_ref[...], wn[...]) + bn[...])
    nf_ref[...] = nf
    ps_ref[...] = _dot(nf, ws[...])
    pd_ref[...] = _dot(nf, wd[...])


def _nodes0(x, Wn0, bn0, Ws, Wd):
    return pl.pallas_call(
        _nodes0_body,
        grid=(N // BN,),
        in_specs=[pl.BlockSpec((BN, D), lambda i: (i, 0)),
                  pl.BlockSpec((D, D), lambda i: (0, 0)),
                  pl.BlockSpec((1, D), lambda i: (0, 0)),
                  pl.BlockSpec((D, H), lambda i: (0, 0)),
                  pl.BlockSpec((D, H), lambda i: (0, 0))],
        out_specs=[pl.BlockSpec((BN, D), lambda i: (i, 0)),
                   pl.BlockSpec((BN, H), lambda i: (i, 0)),
                   pl.BlockSpec((BN, H), lambda i: (i, 0))],
        out_shape=[jax.ShapeDtypeStruct((N, D), jnp.float32),
                   jax.ShapeDtypeStruct((N, H), jnp.float32),
                   jax.ShapeDtypeStruct((N, H), jnp.float32)],
    )(x, Wn0, bn0, Ws, Wd)


def _edge1_body(ea, gs, gd, we0, be0, wme, bm1, wm2, bm2, out_ref):
    ef0 = _relu(_dot(ea[...], we0[...]) + be0[...])
    h = _relu(gs[...] + gd[...] + _dot(ef0, wme[...]) + bm1[...])
    out = _relu(_dot(h, wm2[...]) + bm2[...])
    out_ref[...] = jnp.concatenate(
        [out, jnp.zeros((out.shape[0], DEP - DE), jnp.float32)], axis=1)


def _edge1(ea, gs, gd, We0, be0, Wme, bm1, Wm2, bm2):
    return pl.pallas_call(
        _edge1_body,
        grid=(E // BE,),
        in_specs=[pl.BlockSpec((BE, DE_IN), lambda i: (i, 0)),
                  pl.BlockSpec((BE, H), lambda i: (i, 0)),
                  pl.BlockSpec((BE, H), lambda i: (i, 0)),
                  pl.BlockSpec((DE_IN, DE), lambda i: (0, 0)),
                  pl.BlockSpec((1, DE), lambda i: (0, 0)),
                  pl.BlockSpec((DE, H), lambda i: (0, 0)),
                  pl.BlockSpec((1, H), lambda i: (0, 0)),
                  pl.BlockSpec((H, DE), lambda i: (0, 0)),
                  pl.BlockSpec((1, DE), lambda i: (0, 0))],
        out_specs=pl.BlockSpec((BE, DEP), lambda i: (i, 0)),
        out_shape=jax.ShapeDtypeStruct((E, DEP), jnp.float32),
    )(ea, gs, gd, We0, be0, Wme, bm1, Wm2, bm2)


def _edge2_body(efp, gs, gd, wmep, bm1, wm2, bm2, out_ref):
    h = _relu(gs[...] + gd[...] + _dot(efp[...], wmep[...]) + bm1[...])
    out = _relu(_dot(h, wm2[...]) + bm2[...])
    out_ref[...] = jnp.concatenate(
        [out, jnp.zeros((out.shape[0], DEP - DE), jnp.float32)], axis=1)


def _edge2(efp, gs, gd, Wmep, bm1, Wm2, bm2):
    return pl.pallas_call(
        _edge2_body,
        grid=(E // BE,),
        in_specs=[pl.BlockSpec((BE, DEP), lambda i: (i, 0)),
                  pl.BlockSpec((BE, H), lambda i: (i, 0)),
                  pl.BlockSpec((BE, H), lambda i: (i, 0)),
                  pl.BlockSpec((DEP, H), lambda i: (0, 0)),
                  pl.BlockSpec((1, H), lambda i: (0, 0)),
                  pl.BlockSpec((H, DE), lambda i: (0, 0)),
                  pl.BlockSpec((1, DE), lambda i: (0, 0))],
        out_specs=pl.BlockSpec((BE, DEP), lambda i: (i, 0)),
        out_shape=jax.ShapeDtypeStruct((E, DEP), jnp.float32),
    )(efp, gs, gd, Wmep, bm1, Wm2, bm2)


def _upd_body(nf, a, wux, wua, bu, ws, wd, nf1_ref, ps_ref, pd_ref):
    nf1 = _relu(_dot(nf[...], wux[...]) + _dot(a[...], wua[...]) + bu[...])
    nf1_ref[...] = nf1
    ps_ref[...] = _dot(nf1, ws[...])
    pd_ref[...] = _dot(nf1, wd[...])


def _update_proj(nf, agg, Wux, Wua, bu, Ws, Wd):
    return pl.pallas_call(
        _upd_body,
        grid=(N // BN,),
        in_specs=[pl.BlockSpec((BN, D), lambda i: (i, 0)),
                  pl.BlockSpec((BN, DEP), lambda i: (i, 0)),
                  pl.BlockSpec((D, D), lambda i: (0, 0)),
                  pl.BlockSpec((DEP, D), lambda i: (0, 0)),
                  pl.BlockSpec((1, D), lambda i: (0, 0)),
                  pl.BlockSpec((D, H), lambda i: (0, 0)),
                  pl.BlockSpec((D, H), lambda i: (0, 0))],
        out_specs=[pl.BlockSpec((BN, D), lambda i: (i, 0)),
                   pl.BlockSpec((BN, H), lambda i: (i, 0)),
                   pl.BlockSpec((BN, H), lambda i: (i, 0))],
        out_shape=[jax.ShapeDtypeStruct((N, D), jnp.float32),
                   jax.ShapeDtypeStruct((N, H), jnp.float32),
                   jax.ShapeDtypeStruct((N, H), jnp.float32)],
    )(nf, agg, Wux, Wua, bu, Ws, Wd)


def _head_body(nf, a, wux, wua, bu, wt1, bt1, wt2, bt2, out):
    nf2 = _relu(_dot(nf[...], wux[...]) + _dot(a[...], wua[...]) + bu[...])
    t = _relu(_dot(nf2, wt1[...]) + bt1[...])
    out[...] = jnp.sum(t * wt2[...], axis=1, keepdims=True) + bt2[...]


def _update_head(nf, agg, Wux, Wua, bu, Wt1, bt1, Wt2r, bt2):
    return pl.pallas_call(
        _head_body,
        grid=(N // BN,),
        in_specs=[pl.BlockSpec((BN, D), lambda i: (i, 0)),
                  pl.BlockSpec((BN, DEP), lambda i: (i, 0)),
                  pl.BlockSpec((D, D), lambda i: (0, 0)),
                  pl.BlockSpec((DEP, D), lambda i: (0, 0)),
                  pl.BlockSpec((1, D), lambda i: (0, 0)),
                  pl.BlockSpec((D, DE), lambda i: (0, 0)),
                  pl.BlockSpec((1, DE), lambda i: (0, 0)),
                  pl.BlockSpec((1, DE), lambda i: (0, 0)),
                  pl.BlockSpec((1, 1), lambda i: (0, 0))],
        out_specs=pl.BlockSpec((BN, 1), lambda i: (i, 0)),
        out_shape=jax.ShapeDtypeStruct((N, 1), jnp.float32),
    )(nf, agg, Wux, Wua, bu, Wt1, bt1, Wt2r, bt2)


def kernel(x, edge_attr, edge_index, Wn0, bn0, We0, be0, Wm1, bm1, Wm2, bm2,
           Wu, bu, Wt1, bt1, Wt2, bt2):
    src3 = edge_index[0].astype(jnp.int32).reshape(NW, NCH, CH)
    dst3 = edge_index[1].astype(jnp.int32).reshape(NW, NCH, CH)
    dst2d = edge_index[1].astype(jnp.int32).reshape(_DPR, 128)

    Ws = Wm1[:D]
    Wd = Wm1[D:2 * D]
    Wme = Wm1[2 * D:]
    Wmep = jnp.concatenate([Wme, jnp.zeros((DEP - DE, H), jnp.float32)], axis=0)
    Wux = Wu[:D]
    Wua = Wu[D:]
    Wuap = jnp.concatenate([Wua, jnp.zeros((DEP - DE, D), jnp.float32)], axis=0)
    bn0r = bn0.reshape(1, D)
    be0r = be0.reshape(1, DE)
    bm1r = bm1.reshape(1, H)
    bm2r = bm2.reshape(1, DE)
    bur = bu.reshape(1, D)
    bt1r = bt1.reshape(1, 64)
    Wt2r = Wt2.reshape(1, 64)
    bt2r = bt2.reshape(1, 1)
    zeros = jnp.zeros((CHQ, DEP), jnp.float32)

    sc_gather = _sc_gather_kernel()
    sc_scatter = _sc_scatter_kernel()

    idx4 = _dst_prep(dst2d).reshape(NC, NS, NCHQ, CHQ)

    nf, ps, pd = _nodes0(x, Wn0, bn0r, Ws, Wd)
    gs, gd = sc_gather(ps, pd, src3, dst3)
    efp = _edge1(edge_attr, gs, gd, We0, be0r, Wme, bm1r, Wm2, bm2r)
    agg = sc_scatter(efp, idx4, zeros)
    nf, ps, pd = _update_proj(nf, agg, Wux, Wuap, bur, Ws, Wd)
    gs, gd = sc_gather(ps, pd, src3, dst3)
    efp = _edge2(efp, gs, gd, Wmep, bm1r, Wm2, bm2r)
    agg = sc_scatter(efp, idx4, zeros)
    preds = _update_head(nf, agg, Wux, Wuap, bur, Wt1, bt1r, Wt2r, bt2r)
    return preds.reshape(N)


# speedup vs baseline: 3.7177x; 1.1737x over previous
import functools

import jax
import jax.numpy as jnp
from jax import lax
from jax.experimental import pallas as pl
from jax.experimental.pallas import tpu as pltpu
from jax.experimental.pallas import tpu_sc as plsc

N = 10000
E = 320000
D = 128
DE_IN = 16
DE = 64
H = 128

NC = 2
NS = 16
NW = NC * NS
EPW = E // NW
CH = 80
CPG = 5
GR = CH * CPG
NCH = EPW // CH
NG = NCH // CPG
NPAD = 10240
HR = NPAD // 2
HRP = HR + NS
RPH = HR // NS
EPT = E // NS
DEP = 128
CHQ = 80
NCHQ = EPT // CHQ
DRN = RPH // CHQ

BN = 2000
BE = 4000


@functools.cache
def _sc_mesh():
    return plsc.VectorSubcoreMesh(core_axis_name="c", subcore_axis_name="s",
                                  num_cores=NC, num_subcores=NS)


@functools.cache
def _sc_gather_kernel():
    return pl.kernel(
        _sc_gather_body,
        mesh=_sc_mesh(),
        out_type=[jax.ShapeDtypeStruct((E, H), jnp.float32),
                  jax.ShapeDtypeStruct((E, H), jnp.float32)],
        scratch_types=[[pltpu.VMEM((CH,), jnp.int32) for _ in range(CPG)],
                       [pltpu.VMEM((CH,), jnp.int32) for _ in range(CPG)],
                       pltpu.VMEM((GR, H), jnp.float32),
                       pltpu.VMEM((GR, H), jnp.float32),
                       pltpu.SemaphoreType.DMA,
                       pltpu.SemaphoreType.DMA],
    )


def _sc_gather_body(ps_hbm, pd_hbm, src_hbm, dst_hbm, gs_hbm, gd_hbm,
                    idxa, idxb, bufa, bufb, isem, sem):
    wid = lax.axis_index("s") * NC + lax.axis_index("c")
    base = wid * EPW

    def run(idx_hbm, table, out):
        def idx_start(g, idxs):
            for k in range(CPG):
                pltpu.async_copy(idx_hbm.at[wid, g * CPG + k], idxs[k], isem)

        def idx_wait(g, idxs):
            for k in range(CPG):
                pltpu.make_async_copy(idx_hbm.at[wid, g * CPG + k],
                                      idxs[k], isem).wait()

        def gat_start(idxs, buf):
            for k in range(CPG):
                pltpu.async_copy(table.at[idxs[k]],
                                 buf.at[pl.ds(k * CH, CH)], sem)

        def gat_wait(idxs, buf):
            for k in range(CPG):
                pltpu.make_async_copy(table.at[idxs[k]],
                                      buf.at[pl.ds(k * CH, CH)], sem).wait()

        def write(g, buf):
            pltpu.sync_copy(buf, out.at[pl.ds(base + g * GR, GR)])

        idx_start(0, idxa)
        idx_wait(0, idxa)
        gat_start(idxa, bufa)

        def pair(i, carry):
            g0 = 2 * i
            idx_start(g0 + 1, idxb)
            gat_wait(idxa, bufa)
            idx_wait(g0 + 1, idxb)
            gat_start(idxb, bufb)
            idx_start(g0 + 2, idxa)
            write(g0, bufa)
            gat_wait(idxb, bufb)
            idx_wait(g0 + 2, idxa)
            gat_start(idxa, bufa)
            write(g0 + 1, bufb)
            return carry
        lax.fori_loop(0, (NG - 1) // 2, pair, 0)
        gat_wait(idxa, bufa)
        write(NG - 1, bufa)

    run(src_hbm, ps_hbm, gs_hbm)
    run(dst_hbm, pd_hbm, gd_hbm)


@functools.cache
def _sc_scatter_kernel():
    return pl.kernel(
        _sc_scatter_body,
        mesh=_sc_mesh(),
        out_type=jax.ShapeDtypeStruct((NPAD, DEP), jnp.float32),
        scratch_types=[[pltpu.VMEM((CHQ,), jnp.int32) for _ in range(4)],
                       [pltpu.VMEM((CHQ, DEP), jnp.float32) for _ in range(4)],
                       pltpu.VMEM_SHARED((HRP, DEP), jnp.float32),
                       pltpu.SemaphoreType.DMA,
                       pltpu.SemaphoreType.DMA,
                       pltpu.SemaphoreType.DMA],
    )


def _sc_scatter_body(efp_hbm, idx_hbm, zeros_hbm, agg_hbm, ils,
                     bufs, acc_sh, isem, sem, ssem):
    cid = lax.axis_index("c")
    sid = lax.axis_index("s")
    base = sid * EPT
    pltpu.sync_copy(zeros_hbm, bufs[0])
    for j in range(DRN):
        pltpu.sync_copy(bufs[0], acc_sh.at[pl.ds(sid * RPH + j * CHQ, CHQ)])
    plsc.subcore_barrier()

    def load(g, b):
        pltpu.async_copy(idx_hbm.at[cid, sid, g], ils[b], isem)
        pltpu.async_copy(efp_hbm.at[pl.ds(base + g * CHQ, CHQ)], bufs[b], sem)

    def wait_load(g, b):
        pltpu.make_async_copy(idx_hbm.at[cid, sid, g], ils[b], isem).wait()
        pltpu.make_async_copy(efp_hbm.at[pl.ds(base + g * CHQ, CHQ)],
                              bufs[b], sem).wait()

    for b in range(4):
        load(b, b)

    def quad(i, carry):
        for k in range(4):
            wait_load(4 * i + k, k)
            pltpu.async_copy(bufs[k], acc_sh.at[ils[k]], ssem, add=True)
        for k in range(4):
            g = 4 * i + k
            pltpu.make_async_copy(bufs[k], acc_sh.at[ils[k]], ssem).wait()
            @pl.when(g + 4 < NCHQ)
            def _():
                load(g + 4, k)
        return carry
    lax.fori_loop(0, NCHQ // 4, quad, 0)
    for k in range(NCHQ % 4):
        wait_load(NCHQ - NCHQ % 4 + k, k)
        pltpu.async_copy(bufs[k], acc_sh.at[ils[k]], ssem, add=True)
    for k in range(NCHQ % 4):
        pltpu.make_async_copy(bufs[k], acc_sh.at[ils[k]], ssem).wait()

    plsc.subcore_barrier()
    for j in range(DRN):
        pltpu.sync_copy(acc_sh.at[pl.ds(sid * RPH + j * CHQ, CHQ)], bufs[0])
        pltpu.sync_copy(bufs[0],
                        agg_hbm.at[pl.ds(cid * HR + sid * RPH + j * CHQ, CHQ)])


def _relu(v):
    return jnp.maximum(v, 0.0)


_DPR = E // 128
_DBR = _DPR


def _dst_prep_body(d_ref, o_ref):
    c = pl.program_id(0)
    i = pl.program_id(1)
    v = d_ref[...] - c * HR
    ok = (v >= 0) & (v < HR)
    rows = lax.broadcasted_iota(jnp.int32, (_DBR, 128), 0)
    cols = lax.broadcasted_iota(jnp.int32, (_DBR, 128), 1)
    e = (i * _DBR + rows) * 128 + cols
    trash = HR + e // EPT
    o_ref[...] = jnp.where(ok, v, trash)[None]


def _dst_prep(dst2d):
    return pl.pallas_call(
        _dst_prep_body,
        grid=(NC, _DPR // _DBR),
        in_specs=[pl.BlockSpec((_DBR, 128), lambda c, i: (i, 0))],
        out_specs=pl.BlockSpec((1, _DBR, 128), lambda c, i: (c, i, 0)),
        out_shape=jax.ShapeDtypeStruct((NC, _DPR, 128), jnp.int32),
    )(dst2d)


def _dot(a, b):
    return lax.dot_general(a, b, (((1,), (0,)), ((), ())),
                           preferred_element_type=jnp.float32)


def _nodes0_body(x_ref, wn, bn, ws, wd, nf_ref, ps_ref, pd_ref):
    nf = _relu(_dot(x_ref[...], wn[...]) + bn[...])
    nf_ref[...] = nf
    ps_ref[...] = _dot(nf, ws[...])
    pd_ref[...] = _dot(nf, wd[...])


def _nodes0(x, Wn0, bn0, Ws, Wd):
    return pl.pallas_call(
        _nodes0_body,
        grid=(N // BN,),
        in_specs=[pl.BlockSpec((BN, D), lambda i: (i, 0)),
                  pl.BlockSpec((D, D), lambda i: (0, 0)),
                  pl.BlockSpec((1, D), lambda i: (0, 0)),
                  pl.BlockSpec((D, H), lambda i: (0, 0)),
                  pl.BlockSpec((D, H), lambda i: (0, 0))],
        out_specs=[pl.BlockSpec((BN, D), lambda i: (i, 0)),
                   pl.BlockSpec((BN, H), lambda i: (i, 0)),
                   pl.BlockSpec((BN, H), lambda i: (i, 0))],
        out_shape=[jax.ShapeDtypeStruct((N, D), jnp.float32),
                   jax.ShapeDtypeStruct((N, H), jnp.float32),
                   jax.ShapeDtypeStruct((N, H), jnp.float32)],
    )(x, Wn0, bn0, Ws, Wd)


def _edge1_body(ea, gs, gd, we0, be0, wme, bm1, wm2, bm2, out_ref):
    ef0 = _relu(_dot(ea[...], we0[...]) + be0[...])
    h = _relu(gs[...] + gd[...] + _dot(ef0, wme[...]) + bm1[...])
    out = _relu(_dot(h, wm2[...]) + bm2[...])
    out_ref[...] = jnp.concatenate(
        [out, jnp.zeros((out.shape[0], DEP - DE), jnp.float32)], axis=1)


def _edge1(ea, gs, gd, We0, be0, Wme, bm1, Wm2, bm2):
    return pl.pallas_call(
        _edge1_body,
        grid=(E // BE,),
        in_specs=[pl.BlockSpec((BE, DE_IN), lambda i: (i, 0)),
                  pl.BlockSpec((BE, H), lambda i: (i, 0)),
                  pl.BlockSpec((BE, H), lambda i: (i, 0)),
                  pl.BlockSpec((DE_IN, DE), lambda i: (0, 0)),
                  pl.BlockSpec((1, DE), lambda i: (0, 0)),
                  pl.BlockSpec((DE, H), lambda i: (0, 0)),
                  pl.BlockSpec((1, H), lambda i: (0, 0)),
                  pl.BlockSpec((H, DE), lambda i: (0, 0)),
                  pl.BlockSpec((1, DE), lambda i: (0, 0))],
        out_specs=pl.BlockSpec((BE, DEP), lambda i: (i, 0)),
        out_shape=jax.ShapeDtypeStruct((E, DEP), jnp.float32),
    )(ea, gs, gd, We0, be0, Wme, bm1, Wm2, bm2)


def _edge2_body(efp, gs, gd, wmep, bm1, wm2, bm2, out_ref):
    h = _relu(gs[...] + gd[...] + _dot(efp[...], wmep[...]) + bm1[...])
    out = _relu(_dot(h, wm2[...]) + bm2[...])
    out_ref[...] = jnp.concatenate(
        [out, jnp.zeros((out.shape[0], DEP - DE), jnp.float32)], axis=1)


def _edge2(efp, gs, gd, Wmep, bm1, Wm2, bm2):
    return pl.pallas_call(
        _edge2_body,
        grid=(E // BE,),
        in_specs=[pl.BlockSpec((BE, DEP), lambda i: (i, 0)),
                  pl.BlockSpec((BE, H), lambda i: (i, 0)),
                  pl.BlockSpec((BE, H), lambda i: (i, 0)),
                  pl.BlockSpec((DEP, H), lambda i: (0, 0)),
                  pl.BlockSpec((1, H), lambda i: (0, 0)),
                  pl.BlockSpec((H, DE), lambda i: (0, 0)),
                  pl.BlockSpec((1, DE), lambda i: (0, 0))],
        out_specs=pl.BlockSpec((BE, DEP), lambda i: (i, 0)),
        out_shape=jax.ShapeDtypeStruct((E, DEP), jnp.float32),
    )(efp, gs, gd, Wmep, bm1, Wm2, bm2)


def _upd_body(nf, a, wux, wua, bu, ws, wd, nf1_ref, ps_ref, pd_ref):
    nf1 = _relu(_dot(nf[...], wux[...]) + _dot(a[...], wua[...]) + bu[...])
    nf1_ref[...] = nf1
    ps_ref[...] = _dot(nf1, ws[...])
    pd_ref[...] = _dot(nf1, wd[...])


def _update_proj(nf, agg, Wux, Wua, bu, Ws, Wd):
    return pl.pallas_call(
        _upd_body,
        grid=(N // BN,),
        in_specs=[pl.BlockSpec((BN, D), lambda i: (i, 0)),
                  pl.BlockSpec((BN, DEP), lambda i: (i, 0)),
                  pl.BlockSpec((D, D), lambda i: (0, 0)),
                  pl.BlockSpec((DEP, D), lambda i: (0, 0)),
                  pl.BlockSpec((1, D), lambda i: (0, 0)),
                  pl.BlockSpec((D, H), lambda i: (0, 0)),
                  pl.BlockSpec((D, H), lambda i: (0, 0))],
        out_specs=[pl.BlockSpec((BN, D), lambda i: (i, 0)),
                   pl.BlockSpec((BN, H), lambda i: (i, 0)),
                   pl.BlockSpec((BN, H), lambda i: (i, 0))],
        out_shape=[jax.ShapeDtypeStruct((N, D), jnp.float32),
                   jax.ShapeDtypeStruct((N, H), jnp.float32),
                   jax.ShapeDtypeStruct((N, H), jnp.float32)],
    )(nf, agg, Wux, Wua, bu, Ws, Wd)


def _head_body(nf, a, wux, wua, bu, wt1, bt1, wt2, bt2, out):
    nf2 = _relu(_dot(nf[...], wux[...]) + _dot(a[...], wua[...]) + bu[...])
    t = _relu(_dot(nf2, wt1[...]) + bt1[...])
    out[...] = jnp.sum(t * wt2[...], axis=1, keepdims=True) + bt2[...]


def _update_head(nf, agg, Wux, Wua, bu, Wt1, bt1, Wt2r, bt2):
    return pl.pallas_call(
        _head_body,
        grid=(N // BN,),
        in_specs=[pl.BlockSpec((BN, D), lambda i: (i, 0)),
                  pl.BlockSpec((BN, DEP), lambda i: (i, 0)),
                  pl.BlockSpec((D, D), lambda i: (0, 0)),
                  pl.BlockSpec((DEP, D), lambda i: (0, 0)),
                  pl.BlockSpec((1, D), lambda i: (0, 0)),
                  pl.BlockSpec((D, DE), lambda i: (0, 0)),
                  pl.BlockSpec((1, DE), lambda i: (0, 0)),
                  pl.BlockSpec((1, DE), lambda i: (0, 0)),
                  pl.BlockSpec((1, 1), lambda i: (0, 0))],
        out_specs=pl.BlockSpec((BN, 1), lambda i: (i, 0)),
        out_shape=jax.ShapeDtypeStruct((N, 1), jnp.float32),
    )(nf, agg, Wux, Wua, bu, Wt1, bt1, Wt2r, bt2)


def kernel(x, edge_attr, edge_index, Wn0, bn0, We0, be0, Wm1, bm1, Wm2, bm2,
           Wu, bu, Wt1, bt1, Wt2, bt2):
    src3 = edge_index[0].astype(jnp.int32).reshape(NW, NCH, CH)
    dst3 = edge_index[1].astype(jnp.int32).reshape(NW, NCH, CH)
    dst2d = edge_index[1].astype(jnp.int32).reshape(_DPR, 128)

    Ws = Wm1[:D]
    Wd = Wm1[D:2 * D]
    Wme = Wm1[2 * D:]
    Wmep = jnp.concatenate([Wme, jnp.zeros((DEP - DE, H), jnp.float32)], axis=0)
    Wux = Wu[:D]
    Wua = Wu[D:]
    Wuap = jnp.concatenate([Wua, jnp.zeros((DEP - DE, D), jnp.float32)], axis=0)
    bn0r = bn0.reshape(1, D)
    be0r = be0.reshape(1, DE)
    bm1r = bm1.reshape(1, H)
    bm2r = bm2.reshape(1, DE)
    bur = bu.reshape(1, D)
    bt1r = bt1.reshape(1, 64)
    Wt2r = Wt2.reshape(1, 64)
    bt2r = bt2.reshape(1, 1)
    zeros = jnp.zeros((CHQ, DEP), jnp.float32)

    sc_gather = _sc_gather_kernel()
    sc_scatter = _sc_scatter_kernel()

    idx4 = _dst_prep(dst2d).reshape(NC, NS, NCHQ, CHQ)

    nf, ps, pd = _nodes0(x, Wn0, bn0r, Ws, Wd)
    gs, gd = sc_gather(ps, pd, src3, dst3)
    efp = _edge1(edge_attr, gs, gd, We0, be0r, Wme, bm1r, Wm2, bm2r)
    agg = sc_scatter(efp, idx4, zeros)
    nf, ps, pd = _update_proj(nf, agg, Wux, Wuap, bur, Ws, Wd)
    gs, gd = sc_gather(ps, pd, src3, dst3)
    efp = _edge2(efp, gs, gd, Wmep, bm1r, Wm2, bm2r)
    agg = sc_scatter(efp, idx4, zeros)
    preds = _update_head(nf, agg, Wux, Wuap, bur, Wt1, bt1r, Wt2r, bt2r)
    return preds.reshape(N)
